# Initial kernel scaffold; baseline (speedup 1.0000x reference)
import functools

import jax
import jax.numpy as jnp
from jax import lax
from jax.experimental import pallas as pl
from jax.experimental.pallas import tpu as pltpu
from jax.experimental.pallas import tpu_sc as plsc

N = 10000
NP = 10240
D = 128
E = 320000
NCORES = 2
NSUB = 16
NW = NCORES * NSUB
CH = 128
NCHUNK = 80
EPAD = NW * NCHUNK * CH
ROWS_PER_TILE = NP // NSUB
NCOPY = ROWS_PER_TILE // CH

_mesh = plsc.VectorSubcoreMesh(
    core_axis_name="c", subcore_axis_name="s",
    num_cores=NCORES, num_subcores=NSUB)


@functools.partial(
    pl.kernel,
    out_type=jax.ShapeDtypeStruct((NCORES, NP, 16), jnp.float32),
    mesh=_mesh,
    scratch_types=[
        pltpu.VMEM((NCHUNK, CH), jnp.int32),
        pltpu.VMEM((CH, 16), jnp.float32),
        pltpu.VMEM((CH, 16), jnp.float32),
        pltpu.VMEM_SHARED((NP, 16), jnp.float32),
    ],
)
def _deg_kernel(dst_hbm, out_hbm, idx_v, ones_v, zeros_v, deg_sh):
    c = lax.axis_index("c")
    s = lax.axis_index("s")
    wid = c * NSUB + s

    def fill(i, _):
        ones_v[i, :] = jnp.full((16,), 1.0, jnp.float32)
        zeros_v[i, :] = jnp.zeros((16,), jnp.float32)
        return 0
    lax.fori_loop(0, CH, fill, 0)

    base = s * ROWS_PER_TILE
    for k in range(NCOPY):
        pltpu.sync_copy(zeros_v, deg_sh.at[pl.ds(base + k * CH, CH)])
    pltpu.sync_copy(dst_hbm.at[wid], idx_v)
    plsc.subcore_barrier()

    def body(j, _):
        pltpu.sync_copy(ones_v, deg_sh.at[idx_v.at[j]], add=True)
        return 0
    lax.fori_loop(0, NCHUNK, body, 0)

    plsc.subcore_barrier()
    for k in range(NCOPY):
        pltpu.sync_copy(deg_sh.at[pl.ds(base + k * CH, CH)],
                        out_hbm.at[c, pl.ds(base + k * CH, CH)])


@functools.partial(
    pl.kernel,
    out_type=jax.ShapeDtypeStruct((NCORES, NP, D), jnp.float32),
    mesh=_mesh,
    scratch_types=[
        pltpu.VMEM((NCHUNK, CH), jnp.int32),
        pltpu.VMEM((NCHUNK, CH), jnp.int32),
        pltpu.VMEM((CH, D), jnp.float32),
        pltpu.VMEM((CH, D), jnp.float32),
        pltpu.VMEM((CH, D), jnp.float32),
        pltpu.VMEM_SHARED((NP, D), jnp.float32),
        pltpu.SemaphoreType.DMA,
        pltpu.SemaphoreType.DMA,
    ],
)
def _agg_kernel(hs_hbm, src_hbm, dst_hbm, out_hbm,
                src_v, dst_v, buf0, buf1, zeros_v, agg_sh, sem0, sem1):
    c = lax.axis_index("c")
    s = lax.axis_index("s")
    wid = c * NSUB + s

    def fill(i, _):
        for q in range(D // 16):
            zeros_v[i, pl.ds(q * 16, 16)] = jnp.zeros((16,), jnp.float32)
        return 0
    lax.fori_loop(0, CH, fill, 0)

    base = s * ROWS_PER_TILE
    for k in range(NCOPY):
        pltpu.sync_copy(zeros_v, agg_sh.at[pl.ds(base + k * CH, CH)])
    pltpu.sync_copy(src_hbm.at[wid], src_v)
    pltpu.sync_copy(dst_hbm.at[wid], dst_v)
    plsc.subcore_barrier()

    bufs = (buf0, buf1)
    sems = (sem0, sem1)
    pltpu.async_copy(hs_hbm.at[src_v.at[0]], buf0, sem0)
    pltpu.async_copy(hs_hbm.at[src_v.at[1]], buf1, sem1)

    def body(g, _):
        for b in range(2):
            j = 2 * g + b
            pltpu.make_async_copy(hs_hbm.at[src_v.at[j]], bufs[b], sems[b]).wait()
            pltpu.sync_copy(bufs[b], agg_sh.at[dst_v.at[j]], add=True)

            @pl.when(j + 2 < NCHUNK)
            def _():
                pltpu.async_copy(hs_hbm.at[src_v.at[j + 2]], bufs[b], sems[b])
        return 0
    lax.fori_loop(0, NCHUNK // 2, body, 0)

    plsc.subcore_barrier()
    for k in range(NCOPY):
        pltpu.sync_copy(agg_sh.at[pl.ds(base + k * CH, CH)],
                        out_hbm.at[c, pl.ds(base + k * CH, CH)])


BLK = 1024
GRID = NP // BLK


def _dinv(deg0_ref, deg1_ref):
    deg = deg0_ref[:, 0:1] + deg1_ref[:, 0:1] + 1.0
    return lax.rsqrt(deg)


def _tc1_body(deg0_ref, deg1_ref, x_ref, w_ref, out_ref):
    dinv = _dinv(deg0_ref, deg1_ref)
    h = jnp.dot(x_ref[...], w_ref[...], preferred_element_type=jnp.float32)
    out_ref[...] = h * dinv


def _tc2_body(deg0_ref, deg1_ref, a0_ref, a1_ref, hs_ref, w_ref, b_ref,
              out_ref):
    dinv = _dinv(deg0_ref, deg1_ref)
    o1 = (a0_ref[...] + a1_ref[...] + hs_ref[...]) * dinv + b_ref[...]
    r = jnp.maximum(o1, 0.0)
    h2 = jnp.dot(r, w_ref[...], preferred_element_type=jnp.float32)
    out_ref[...] = h2 * dinv


def _tc3_body(deg0_ref, deg1_ref, a0_ref, a1_ref, hs_ref, b_ref, out_ref):
    dinv = _dinv(deg0_ref, deg1_ref)
    out_ref[...] = (a0_ref[...] + a1_ref[...] + hs_ref[...]) * dinv + b_ref[...]


def _spec_rows(width):
    return pl.BlockSpec((BLK, width), lambda i: (i, 0))


_tc1 = pl.pallas_call(
    _tc1_body,
    grid=(GRID,),
    in_specs=[
        _spec_rows(16), _spec_rows(16), _spec_rows(D),
        pl.BlockSpec((D, D), lambda i: (0, 0)),
    ],
    out_specs=_spec_rows(D),
    out_shape=jax.ShapeDtypeStruct((NP, D), jnp.float32),
)

_tc2 = pl.pallas_call(
    _tc2_body,
    grid=(GRID,),
    in_specs=[
        _spec_rows(16), _spec_rows(16), _spec_rows(D), _spec_rows(D),
        _spec_rows(D),
        pl.BlockSpec((D, D), lambda i: (0, 0)),
        pl.BlockSpec((1, D), lambda i: (0, 0)),
    ],
    out_specs=_spec_rows(D),
    out_shape=jax.ShapeDtypeStruct((NP, D), jnp.float32),
)

_tc3 = pl.pallas_call(
    _tc3_body,
    grid=(GRID,),
    in_specs=[
        _spec_rows(16), _spec_rows(16), _spec_rows(D), _spec_rows(D),
        _spec_rows(D),
        pl.BlockSpec((1, D), lambda i: (0, 0)),
    ],
    out_specs=_spec_rows(D),
    out_shape=jax.ShapeDtypeStruct((NP, D), jnp.float32),
)


def kernel(x, edge_index, W1, b1, W2, b2):
    src = edge_index[0].astype(jnp.int32)
    dst = edge_index[1].astype(jnp.int32)
    pad = jnp.full((EPAD - E,), N, jnp.int32)
    srcs = jnp.concatenate([src, pad]).reshape(NW, NCHUNK, CH)
    dsts = jnp.concatenate([dst, pad]).reshape(NW, NCHUNK, CH)
    xp = jnp.pad(x, ((0, NP - N), (0, 0)))

    degs = _deg_kernel(dsts)
    deg0, deg1 = degs[0], degs[1]

    hs1 = _tc1(deg0, deg1, xp, W1)
    agg1 = _agg_kernel(hs1, srcs, dsts)
    hs2 = _tc2(deg0, deg1, agg1[0], agg1[1], hs1, W2, b1.reshape(1, D))
    agg2 = _agg_kernel(hs2, srcs, dsts)
    out = _tc3(deg0, deg1, agg2[0], agg2[1], hs2, b2.reshape(1, D))
    return out[:N]


# speedup vs baseline: 9.8616x; 9.8616x over previous
import functools

import jax
import jax.numpy as jnp
from jax import lax
from jax.experimental import pallas as pl
from jax.experimental.pallas import tpu as pltpu
from jax.experimental.pallas import tpu_sc as plsc

N = 10000
NP = 10240
D = 128
E = 320000
NCORES = 2
NSUB = 16
NW = NCORES * NSUB
CH = 128
NCHUNK = 80
NSLAB = 2
SLAB = NCHUNK // NSLAB
EPAD = NW * NCHUNK * CH
ROWS_PER_TILE = NP // NSUB
NCOPY = ROWS_PER_TILE // CH

_mesh = plsc.VectorSubcoreMesh(
    core_axis_name="c", subcore_axis_name="s",
    num_cores=NCORES, num_subcores=NSUB)


@functools.partial(
    pl.kernel,
    out_type=jax.ShapeDtypeStruct((NCORES, NP, 16), jnp.float32),
    mesh=_mesh,
    scratch_types=[
        pltpu.VMEM((NCHUNK, CH), jnp.int32),
        pltpu.VMEM((CH, 16), jnp.float32),
        pltpu.VMEM((CH, 16), jnp.float32),
        pltpu.VMEM_SHARED((NP, 16), jnp.float32),
    ],
)
def _deg_kernel(dst_hbm, out_hbm, idx_v, ones_v, zeros_v, deg_sh):
    c = lax.axis_index("c")
    s = lax.axis_index("s")
    wid = c * NSUB + s

    def fill(i, _):
        ones_v[i, :] = jnp.full((16,), 1.0, jnp.float32)
        zeros_v[i, :] = jnp.zeros((16,), jnp.float32)
        return 0
    lax.fori_loop(0, CH, fill, 0)

    base = s * ROWS_PER_TILE
    for k in range(NCOPY):
        pltpu.sync_copy(zeros_v, deg_sh.at[pl.ds(base + k * CH, CH)])
    pltpu.sync_copy(dst_hbm.at[wid], idx_v)
    plsc.subcore_barrier()

    def body(j, _):
        pltpu.sync_copy(ones_v, deg_sh.at[idx_v.at[j]], add=True)
        return 0
    lax.fori_loop(0, NCHUNK, body, 0)

    plsc.subcore_barrier()
    for k in range(NCOPY):
        pltpu.sync_copy(deg_sh.at[pl.ds(base + k * CH, CH)],
                        out_hbm.at[c, pl.ds(base + k * CH, CH)])


@functools.partial(
    pl.kernel,
    out_type=jax.ShapeDtypeStruct((NCORES, NP, D), jnp.float32),
    mesh=_mesh,
    scratch_types=[
        pltpu.VMEM((SLAB, CH), jnp.int32),
        pltpu.VMEM((SLAB, CH), jnp.int32),
        pltpu.VMEM((CH, D), jnp.float32),
        pltpu.VMEM((CH, D), jnp.float32),
        pltpu.VMEM_SHARED((NP, D), jnp.float32),
        pltpu.SemaphoreType.DMA,
        pltpu.SemaphoreType.DMA,
    ],
)
def _agg_kernel(hs_hbm, src_hbm, dst_hbm, out_hbm,
                src_v, dst_v, buf0, buf1, agg_sh, sem0, sem1):
    c = lax.axis_index("c")
    s = lax.axis_index("s")
    wid = c * NSUB + s

    def fill(i, _):
        for q in range(D // 16):
            buf0[i, pl.ds(q * 16, 16)] = jnp.zeros((16,), jnp.float32)
        return 0
    lax.fori_loop(0, CH, fill, 0)

    base = s * ROWS_PER_TILE
    for k in range(NCOPY):
        pltpu.sync_copy(buf0, agg_sh.at[pl.ds(base + k * CH, CH)])
    plsc.subcore_barrier()

    bufs = (buf0, buf1)
    sems = (sem0, sem1)
    for p in range(NSLAB):
        pltpu.sync_copy(src_hbm.at[wid, pl.ds(p * SLAB, SLAB)], src_v)
        pltpu.sync_copy(dst_hbm.at[wid, pl.ds(p * SLAB, SLAB)], dst_v)
        pltpu.async_copy(hs_hbm.at[src_v.at[0]], buf0, sem0)
        pltpu.async_copy(hs_hbm.at[src_v.at[1]], buf1, sem1)

        def body(g, _):
            for b in range(2):
                j = 2 * g + b
                pltpu.make_async_copy(
                    hs_hbm.at[src_v.at[j]], bufs[b], sems[b]).wait()
                pltpu.sync_copy(bufs[b], agg_sh.at[dst_v.at[j]], add=True)

                @pl.when(j + 2 < SLAB)
                def _():
                    pltpu.async_copy(hs_hbm.at[src_v.at[j + 2]], bufs[b], sems[b])
            return 0
        lax.fori_loop(0, SLAB // 2, body, 0)

    plsc.subcore_barrier()
    for k in range(NCOPY):
        pltpu.sync_copy(agg_sh.at[pl.ds(base + k * CH, CH)],
                        out_hbm.at[c, pl.ds(base + k * CH, CH)])


BLK = 1024
GRID = NP // BLK


def _dinv(deg0_ref, deg1_ref):
    deg = deg0_ref[:, 0:1] + deg1_ref[:, 0:1] + 1.0
    return lax.rsqrt(deg)


def _tc1_body(deg0_ref, deg1_ref, x_ref, w_ref, out_ref):
    dinv = _dinv(deg0_ref, deg1_ref)
    h = jnp.dot(x_ref[...], w_ref[...], preferred_element_type=jnp.float32)
    out_ref[...] = h * dinv


def _tc2_body(deg0_ref, deg1_ref, a0_ref, a1_ref, hs_ref, w_ref, b_ref,
              out_ref):
    dinv = _dinv(deg0_ref, deg1_ref)
    o1 = (a0_ref[...] + a1_ref[...] + hs_ref[...]) * dinv + b_ref[...]
    r = jnp.maximum(o1, 0.0)
    h2 = jnp.dot(r, w_ref[...], preferred_element_type=jnp.float32)
    out_ref[...] = h2 * dinv


def _tc3_body(deg0_ref, deg1_ref, a0_ref, a1_ref, hs_ref, b_ref, out_ref):
    dinv = _dinv(deg0_ref, deg1_ref)
    out_ref[...] = (a0_ref[...] + a1_ref[...] + hs_ref[...]) * dinv + b_ref[...]


def _spec_rows(width):
    return pl.BlockSpec((BLK, width), lambda i: (i, 0))


_tc1 = pl.pallas_call(
    _tc1_body,
    grid=(GRID,),
    in_specs=[
        _spec_rows(16), _spec_rows(16), _spec_rows(D),
        pl.BlockSpec((D, D), lambda i: (0, 0)),
    ],
    out_specs=_spec_rows(D),
    out_shape=jax.ShapeDtypeStruct((NP, D), jnp.float32),
)

_tc2 = pl.pallas_call(
    _tc2_body,
    grid=(GRID,),
    in_specs=[
        _spec_rows(16), _spec_rows(16), _spec_rows(D), _spec_rows(D),
        _spec_rows(D),
        pl.BlockSpec((D, D), lambda i: (0, 0)),
        pl.BlockSpec((1, D), lambda i: (0, 0)),
    ],
    out_specs=_spec_rows(D),
    out_shape=jax.ShapeDtypeStruct((NP, D), jnp.float32),
)

_tc3 = pl.pallas_call(
    _tc3_body,
    grid=(GRID,),
    in_specs=[
        _spec_rows(16), _spec_rows(16), _spec_rows(D), _spec_rows(D),
        _spec_rows(D),
        pl.BlockSpec((1, D), lambda i: (0, 0)),
    ],
    out_specs=_spec_rows(D),
    out_shape=jax.ShapeDtypeStruct((NP, D), jnp.float32),
)


def kernel(x, edge_index, W1, b1, W2, b2):
    src = edge_index[0].astype(jnp.int32)
    dst = edge_index[1].astype(jnp.int32)
    pad = jnp.full((EPAD - E,), N, jnp.int32)
    srcs = jnp.concatenate([src, pad]).reshape(NW, NCHUNK, CH)
    dsts = jnp.concatenate([dst, pad]).reshape(NW, NCHUNK, CH)
    xp = jnp.pad(x, ((0, NP - N), (0, 0)))

    degs = _deg_kernel(dsts)
    deg0, deg1 = degs[0], degs[1]

    hs1 = _tc1(deg0, deg1, xp, W1)
    agg1 = _agg_kernel(hs1, srcs, dsts)
    hs2 = _tc2(deg0, deg1, agg1[0], agg1[1], hs1, W2, b1.reshape(1, D))
    agg2 = _agg_kernel(hs2, srcs, dsts)
    out = _tc3(deg0, deg1, agg2[0], agg2[1], hs2, b2.reshape(1, D))
    return out[:N]
